# Initial kernel scaffold; baseline (speedup 1.0000x reference)
import jax
import jax.numpy as jnp
from jax.experimental import pallas as pl


def kernel(input_labels, pos_labels, neg_labels, in_table, out_table):
    raise NotImplementedError("write your pallas kernel here")


# speedup vs baseline: 9.7757x; 9.7757x over previous
import functools

import jax
import jax.numpy as jnp
from jax import lax
from jax.experimental import pallas as pl
from jax.experimental.pallas import tpu as pltpu
from jax.experimental.pallas import tpu_sc as plsc

NC = 2
NS = 16
NW = NC * NS

G = 8


def _sc_gather(labels, input_labels, out_table, in_table, B, J, H):
    bpw = B // NW
    mesh = plsc.VectorSubcoreMesh(core_axis_name="c", subcore_axis_name="s")

    @functools.partial(
        pl.kernel,
        out_type=(
            jax.ShapeDtypeStruct((B, J, H), jnp.float32),
            jax.ShapeDtypeStruct((B, H), jnp.float32),
        ),
        mesh=mesh,
        compiler_params=pltpu.CompilerParams(use_tc_tiling_on_sc=False),
        scratch_types=[
            pltpu.VMEM((G, J), jnp.int32),
            pltpu.VMEM((G, J, H), jnp.float32),
            pltpu.VMEM((G,), jnp.int32),
            pltpu.VMEM((G, H), jnp.float32),
            pltpu.SemaphoreType.DMA,
            pltpu.SemaphoreType.DMA,
        ],
    )
    def gather_kernel(labels_hbm, inlab_hbm, outtab_hbm, intab_hbm,
                      rows_hbm, inrows_hbm,
                      idx_v, rows_v, idx2_v, inrows_v, sem, sem2):
        wid = lax.axis_index("s") * NC + lax.axis_index("c")
        base_b = wid * bpw

        @pl.loop(0, bpw, step=G)
        def _(cb):
            b0 = base_b + cb
            pltpu.sync_copy(labels_hbm.at[pl.ds(b0, G)], idx_v)
            pltpu.sync_copy(inlab_hbm.at[pl.ds(b0, G)], idx2_v)
            cps = [
                pltpu.async_copy(outtab_hbm.at[idx_v.at[g]], rows_v.at[g], sem)
                for g in range(G)
            ]
            cin = pltpu.async_copy(intab_hbm.at[idx2_v], inrows_v, sem2)
            for c in cps:
                c.wait()
            cin.wait()
            pltpu.sync_copy(rows_v, rows_hbm.at[pl.ds(b0, G)])
            pltpu.sync_copy(inrows_v, inrows_hbm.at[pl.ds(b0, G)])

    return gather_kernel(labels, input_labels, out_table, in_table)


def _tc_loss(rows, in_rows, B, J, P):
    BB = 128

    def body(rows_ref, emb_ref, o_ref):
        r = rows_ref[...]
        e = emb_ref[...]
        d = jnp.sum(r * e[:, None, :], axis=2)
        j = lax.broadcasted_iota(jnp.int32, (BB, J), 1)
        x = jnp.where(j < P, d, -d) + 1e-9
        ls = jnp.minimum(x, 0.0) - jnp.log1p(jnp.exp(-jnp.abs(x)))
        o_ref[...] = -jnp.sum(ls, axis=1)

    H = rows.shape[-1]
    return pl.pallas_call(
        body,
        grid=(B // BB,),
        in_specs=[
            pl.BlockSpec((BB, J, H), lambda i: (i, 0, 0)),
            pl.BlockSpec((BB, H), lambda i: (i, 0)),
        ],
        out_specs=pl.BlockSpec((BB,), lambda i: (i,)),
        out_shape=jax.ShapeDtypeStruct((B,), jnp.float32),
    )(rows, in_rows)


def kernel(input_labels, pos_labels, neg_labels, in_table, out_table):
    B = input_labels.shape[0]
    P = pos_labels.shape[1]
    N = neg_labels.shape[1]
    H = in_table.shape[1]
    J = P + N

    labels = jnp.concatenate([pos_labels, neg_labels], axis=1)
    rows, in_rows = _sc_gather(labels, input_labels, out_table, in_table, B, J, H)
    return _tc_loss(rows, in_rows, B, J, P)


# speedup vs baseline: 15.1964x; 1.5545x over previous
import functools

import jax
import jax.numpy as jnp
from jax import lax
from jax.experimental import pallas as pl
from jax.experimental.pallas import tpu as pltpu
from jax.experimental.pallas import tpu_sc as plsc

NC = 2
NS = 16
NW = NC * NS

G = 8


def _sc_gather(labels, input_labels, out_table, in_table, B, J, H):
    bpw = B // NW
    mesh = plsc.VectorSubcoreMesh(core_axis_name="c", subcore_axis_name="s")

    @functools.partial(
        pl.kernel,
        out_type=(
            jax.ShapeDtypeStruct((B, J, H), jnp.float32),
            jax.ShapeDtypeStruct((B, H), jnp.float32),
        ),
        mesh=mesh,
        compiler_params=pltpu.CompilerParams(use_tc_tiling_on_sc=False),
        scratch_types=[
            pltpu.VMEM((G, J), jnp.int32),
            pltpu.VMEM((G, J, H), jnp.float32),
            pltpu.VMEM((G,), jnp.int32),
            pltpu.VMEM((G, H), jnp.float32),
            pltpu.SemaphoreType.DMA,
            pltpu.SemaphoreType.DMA,
        ],
    )
    def gather_kernel(labels_hbm, inlab_hbm, outtab_hbm, intab_hbm,
                      rows_hbm, inrows_hbm,
                      idx_v, rows_v, idx2_v, inrows_v, sem, sem2):
        wid = lax.axis_index("s") * NC + lax.axis_index("c")
        base_b = wid * bpw

        @pl.loop(0, bpw, step=G)
        def _(cb):
            b0 = base_b + cb
            pltpu.sync_copy(labels_hbm.at[pl.ds(b0, G)], idx_v)
            pltpu.sync_copy(inlab_hbm.at[pl.ds(b0, G)], idx2_v)
            cps = [
                pltpu.async_copy(outtab_hbm.at[idx_v.at[g]], rows_v.at[g], sem)
                for g in range(G)
            ]
            cin = pltpu.async_copy(intab_hbm.at[idx2_v], inrows_v, sem2)
            for c in cps:
                c.wait()
            cin.wait()
            pltpu.sync_copy(rows_v, rows_hbm.at[pl.ds(b0, G)])
            pltpu.sync_copy(inrows_v, inrows_hbm.at[pl.ds(b0, G)])

    return gather_kernel(labels, input_labels, out_table, in_table)


def _tc_loss(rows2, in_rows, B, J, P, H):
    BB = 256
    GJ = J // 2

    def _logsig(x):
        return jnp.minimum(x, 0.0) - jnp.log1p(jnp.exp(-jnp.abs(x)))

    def body(rows_ref, emb_ref, o_ref):
        r = rows_ref[...].reshape(BB, GJ, 2 * H)
        e = emb_ref[...]
        da = jnp.sum(r[:, :, :H] * e[:, None, :], axis=2)
        db = jnp.sum(r[:, :, H:] * e[:, None, :], axis=2)
        g = lax.broadcasted_iota(jnp.int32, (BB, GJ), 1)
        xa = jnp.where(2 * g < P, da, -da) + 1e-9
        xb = jnp.where(2 * g + 1 < P, db, -db) + 1e-9
        o_ref[...] = -jnp.sum(_logsig(xa) + _logsig(xb), axis=1)

    return pl.pallas_call(
        body,
        grid=(B // BB,),
        in_specs=[
            pl.BlockSpec((BB * GJ, 2 * H), lambda i: (i, 0)),
            pl.BlockSpec((BB, H), lambda i: (i, 0)),
        ],
        out_specs=pl.BlockSpec((BB,), lambda i: (i,)),
        out_shape=jax.ShapeDtypeStruct((B,), jnp.float32),
    )(rows2, in_rows)


def kernel(input_labels, pos_labels, neg_labels, in_table, out_table):
    B = input_labels.shape[0]
    P = pos_labels.shape[1]
    N = neg_labels.shape[1]
    H = in_table.shape[1]
    J = P + N

    labels = jnp.concatenate([pos_labels, neg_labels], axis=1)
    rows, in_rows = _sc_gather(labels, input_labels, out_table, in_table, B, J, H)
    rows2 = rows.reshape(B * J // 2, 2 * H)
    return _tc_loss(rows2, in_rows, B, J, P, H)
